# Initial kernel scaffold; baseline (speedup 1.0000x reference)
import jax
import jax.numpy as jnp
from jax.experimental import pallas as pl


def kernel(x, weights):
    raise NotImplementedError("write your pallas kernel here")


# speedup vs baseline: 1.5398x; 1.5398x over previous
import functools

import jax
import jax.numpy as jnp
from jax import lax
from jax.experimental import pallas as pl
from jax.experimental.pallas import tpu as pltpu
from jax.experimental.pallas import tpu_sc as plsc

_D = 32


@functools.lru_cache(maxsize=None)
def _make_sc_gather(total: int, b_per_w: int, chunk: int):
    info = plsc.get_sparse_core_info()
    nc = info.num_cores
    nchunks = b_per_w // chunk
    mesh = plsc.VectorSubcoreMesh(core_axis_name="c", subcore_axis_name="s")

    @functools.partial(
        pl.kernel,
        mesh=mesh,
        out_type=jax.ShapeDtypeStruct((total, _D), jnp.float32),
        scratch_types=[
            pltpu.VMEM((chunk,), jnp.int32),
            pltpu.VMEM((chunk, _D), jnp.float32),
            pltpu.SemaphoreType.DMA,
        ],
        compiler_params=pltpu.CompilerParams(use_tc_tiling_on_sc=False),
    )
    def k(idx_hbm, table_hbm, out_hbm, idx_v, rows_v, sem):
        wid = lax.axis_index("s") * nc + lax.axis_index("c")
        base = wid * b_per_w

        def body(g, carry):
            start = base + g * chunk
            pltpu.sync_copy(idx_hbm.at[pl.ds(start, chunk)], idx_v)
            pltpu.async_copy(table_hbm.at[idx_v], rows_v, sem).wait()
            pltpu.sync_copy(rows_v, out_hbm.at[pl.ds(start, chunk)])
            return carry

        lax.fori_loop(0, nchunks, body, 0)

    return k


def kernel(x, weights):
    batch, fields = x.shape
    total = batch * fields
    flat_idx = x.reshape(total).astype(jnp.int32)
    b_per_w = total // 32
    chunk = 832
    out = _make_sc_gather(total, b_per_w, chunk)(flat_idx, weights)
    return out.reshape(batch, fields, _D)


# speedup vs baseline: 1.5770x; 1.0241x over previous
import functools

import jax
import jax.numpy as jnp
from jax import lax
from jax.experimental import pallas as pl
from jax.experimental.pallas import tpu as pltpu
from jax.experimental.pallas import tpu_sc as plsc

_D = 32
_NBUF = 4


@functools.lru_cache(maxsize=None)
def _make_sc_gather(total: int, b_per_w: int, chunk: int):
    info = plsc.get_sparse_core_info()
    nc = info.num_cores
    nchunks = b_per_w // chunk
    assert nchunks % _NBUF == 0 and nchunks >= 2 * _NBUF
    mesh = plsc.VectorSubcoreMesh(core_axis_name="c", subcore_axis_name="s")

    @functools.partial(
        pl.kernel,
        mesh=mesh,
        out_type=jax.ShapeDtypeStruct((total, _D), jnp.float32),
        scratch_types=[
            pltpu.VMEM((b_per_w,), jnp.int32),
            pltpu.VMEM((_NBUF, chunk, _D), jnp.float32),
            [pltpu.SemaphoreType.DMA] * _NBUF,
        ],
        compiler_params=pltpu.CompilerParams(use_tc_tiling_on_sc=False),
    )
    def k(idx_hbm, table_hbm, out_hbm, idx_v, rows_v, gsems):
        wid = lax.axis_index("s") * nc + lax.axis_index("c")
        base = wid * b_per_w
        pltpu.sync_copy(idx_hbm.at[pl.ds(base, b_per_w)], idx_v)

        def fire(g, b):
            pltpu.async_copy(
                table_hbm.at[idx_v.at[pl.ds(g * chunk, chunk)]],
                rows_v.at[b],
                gsems[b],
            )

        def drain(g, b):
            pltpu.make_async_copy(
                table_hbm.at[idx_v.at[pl.ds(g * chunk, chunk)]],
                rows_v.at[b],
                gsems[b],
            ).wait()

        for b in range(_NBUF):
            fire(b, b)

        def body(i, carry):
            for b in range(_NBUF):
                g = i * _NBUF + b
                drain(g, b)
                pltpu.sync_copy(rows_v.at[b], out_hbm.at[pl.ds(base + g * chunk, chunk)])
                fire(g + _NBUF, b)
            return carry

        lax.fori_loop(0, nchunks // _NBUF - 1, body, 0)

        for b in range(_NBUF):
            g = nchunks - _NBUF + b
            drain(g, b)
            pltpu.sync_copy(rows_v.at[b], out_hbm.at[pl.ds(base + g * chunk, chunk)])

    return k


def kernel(x, weights):
    batch, fields = x.shape
    total = batch * fields
    flat_idx = x.reshape(total).astype(jnp.int32)
    b_per_w = total // 32
    chunk = 832
    out = _make_sc_gather(total, b_per_w, chunk)(flat_idx, weights)
    return out.reshape(batch, fields, _D)
